# baseline (speedup 1.0000x reference)
import functools

import jax
import jax.numpy as jnp
from jax import lax
from jax.experimental import pallas as pl
from jax.experimental.pallas import tpu as pltpu
from jax.experimental.pallas import tpu_sc as plsc

NTOKEN = 100000
EMB_DIM = 64
OUT_DIM = 2 * EMB_DIM
BATCH = 4096
SEQ = 20
TOT = BATCH * SEQ

NUM_CORES = 2
NUM_SUBCORES = 16
NW = NUM_CORES * NUM_SUBCORES
LANES = 16

NBLK = 782
NTOK_PAD = NBLK * 128
BLKW = 128 * OUT_DIM
NITER = 13


def _fuse_body(at_ref, bt_ref, cat_ref):
    cat_ref[:, 0:EMB_DIM] = at_ref[...].T
    cat_ref[:, EMB_DIM:OUT_DIM] = bt_ref[...].T


_fuse_tables = pl.pallas_call(
    _fuse_body,
    grid=(NBLK,),
    in_specs=[
        pl.BlockSpec((EMB_DIM, 128), lambda g: (0, g)),
        pl.BlockSpec((EMB_DIM, 128), lambda g: (0, g)),
    ],
    out_specs=pl.BlockSpec((128, OUT_DIM), lambda g: (g, 0)),
    out_shape=jax.ShapeDtypeStruct((NTOK_PAD, OUT_DIM), jnp.float32),
    compiler_params=pltpu.CompilerParams(
        dimension_semantics=("arbitrary",)),
)


@functools.partial(
    pl.kernel,
    mesh=plsc.VectorSubcoreMesh(core_axis_name="c", subcore_axis_name="s"),
    out_type=jax.ShapeDtypeStruct((TOT, OUT_DIM), jnp.float32),
    scratch_types=[
        pltpu.VMEM((SEQ, 128), jnp.int32),
        pltpu.VMEM((128, OUT_DIM), jnp.float32),
        pltpu.VMEM((128, OUT_DIM), jnp.float32),
        pltpu.SemaphoreType.DMA,
        pltpu.SemaphoreType.DMA,
        pltpu.SemaphoreType.DMA,
        pltpu.SemaphoreType.DMA,
    ],
)
def _emb_lookup(cat_hbm, xt_hbm, out_hbm, idx_v, r0, r1, sg0, sg1, sw0, sw1):
    wid = lax.axis_index("s") * NUM_CORES + lax.axis_index("c")
    pltpu.sync_copy(xt_hbm.at[:, pl.ds(wid * 128, 128)], idx_v)
    rows = (r0, r1)
    sg = (sg0, sg1)
    sw = (sw0, sw1)
    gathers = [None, None]
    writes = [None, None]
    gathers[0] = pltpu.async_copy(cat_hbm.at[idx_v.at[0]], rows[0], sg[0])
    for s in range(SEQ):
        cur = s % 2
        nxt = (s + 1) % 2
        if s + 1 < SEQ:
            if writes[nxt] is not None:
                writes[nxt].wait()
            gathers[nxt] = pltpu.async_copy(
                cat_hbm.at[idx_v.at[s + 1]], rows[nxt], sg[nxt])
        gathers[cur].wait()
        base = s * BATCH + wid * 128
        writes[cur] = pltpu.async_copy(
            rows[cur], out_hbm.at[pl.ds(base, 128)], sw[cur])
    for w in writes:
        if w is not None:
            w.wait()


def kernel(x, emb_w, embc_w):
    cat_w = _fuse_tables(emb_w.T, embc_w.T)
    out = _emb_lookup(cat_w, x.T)
    out = out.reshape(SEQ, BATCH, OUT_DIM)
    return out.transpose(1, 0, 2)


# speedup vs baseline: 3.0333x; 3.0333x over previous
import functools

import jax
import jax.numpy as jnp
from jax import lax
from jax.experimental import pallas as pl
from jax.experimental.pallas import tpu as pltpu
from jax.experimental.pallas import tpu_sc as plsc

NTOKEN = 100000
EMB_DIM = 64
OUT_DIM = 2 * EMB_DIM
BATCH = 4096
SEQ = 20
TOT = BATCH * SEQ

NUM_CORES = 2
NUM_SUBCORES = 16
NW = NUM_CORES * NUM_SUBCORES
BPW = TOT // NW
CHUNK = 320
NCHUNK = BPW // CHUNK


@functools.partial(
    pl.kernel,
    mesh=plsc.VectorSubcoreMesh(core_axis_name="c", subcore_axis_name="s"),
    out_type=jax.ShapeDtypeStruct((TOT, OUT_DIM), jnp.float32),
    scratch_types=[
        pltpu.VMEM((BPW,), jnp.int32),
        pltpu.VMEM((CHUNK, EMB_DIM), jnp.float32),
        pltpu.VMEM((CHUNK, EMB_DIM), jnp.float32),
        pltpu.VMEM((CHUNK, EMB_DIM), jnp.float32),
        pltpu.VMEM((CHUNK, EMB_DIM), jnp.float32),
        pltpu.SemaphoreType.DMA,
        pltpu.SemaphoreType.DMA,
        pltpu.SemaphoreType.DMA,
        pltpu.SemaphoreType.DMA,
    ],
    compiler_params=pltpu.CompilerParams(use_tc_tiling_on_sc=False),
)
def _emb_lookup(emb_hbm, embc_hbm, x_hbm, out_hbm, idx_v, ra0, rb0, ra1, rb1,
                sg0, sg1, sw0, sw1):
    wid = lax.axis_index("s") * NUM_CORES + lax.axis_index("c")
    pltpu.sync_copy(x_hbm.at[wid], idx_v)
    ra = (ra0, ra1)
    rb = (rb0, rb1)
    sg = (sg0, sg1)
    sw = (sw0, sw1)
    gathers = [None, None]
    writes = [None, None]
    idx0 = idx_v.at[pl.ds(0, CHUNK)]
    gathers[0] = (pltpu.async_copy(emb_hbm.at[idx0], ra[0], sg[0]),
                  pltpu.async_copy(embc_hbm.at[idx0], rb[0], sg[0]))
    for j in range(NCHUNK):
        cur = j % 2
        nxt = (j + 1) % 2
        if j + 1 < NCHUNK:
            if writes[nxt] is not None:
                for w in writes[nxt]:
                    w.wait()
            idx_n = idx_v.at[pl.ds((j + 1) * CHUNK, CHUNK)]
            gathers[nxt] = (
                pltpu.async_copy(emb_hbm.at[idx_n], ra[nxt], sg[nxt]),
                pltpu.async_copy(embc_hbm.at[idx_n], rb[nxt], sg[nxt]),
            )
        for g in gathers[cur]:
            g.wait()
        base = wid * BPW + j * CHUNK
        writes[cur] = (
            pltpu.async_copy(
                ra[cur], out_hbm.at[pl.ds(base, CHUNK), pl.ds(0, EMB_DIM)],
                sw[cur]),
            pltpu.async_copy(
                rb[cur], out_hbm.at[pl.ds(base, CHUNK), pl.ds(EMB_DIM, EMB_DIM)],
                sw[cur]),
        )
    for ws in writes:
        if ws is not None:
            for w in ws:
                w.wait()


def kernel(x, emb_w, embc_w):
    xt = x.T.reshape(NW, BPW)
    out = _emb_lookup(emb_w, embc_w, xt)
    out = out.reshape(SEQ, BATCH, OUT_DIM)
    return out.transpose(1, 0, 2)
